# baseline (speedup 1.0000x reference)
import functools

import jax
import jax.numpy as jnp
from jax import lax
from jax.experimental import pallas as pl
from jax.experimental.pallas import tpu as pltpu
from jax.experimental.pallas import tpu_sc as plsc

S = 512
D = 128
T = 2 * S - 1
NC = 2
NS = 16
NW = NC * NS
RW = S // NW
W = 8
NCH = S // W
SPAN = 528
L = 16
NBUF = 2


def _sc_body(emb_hbm, x_hbm, out_hbm,
             span, xb0, xb1, res0, res1, wsem, csem0, csem1, ssem0, ssem1):
    xbs = (xb0, xb1)
    ress = (res0, res1)
    csems = (csem0, csem1)
    ssems = (ssem0, ssem1)

    wid = lax.axis_index("s") * NC + lax.axis_index("c")
    i0 = wid * RW

    def x_chunk(c):
        return x_hbm.at[pl.ds(pl.multiple_of(c * W, W), W)]

    def out_block(c):
        return out_hbm.at[pl.ds(pl.multiple_of(i0, RW), RW),
                          pl.ds(pl.multiple_of(c * W, W), W)]

    pltpu.make_async_copy(
        emb_hbm.at[pl.ds(pl.multiple_of(i0, RW), SPAN)], span, wsem).start()
    for b in range(NBUF):
        pltpu.make_async_copy(x_chunk(b), xbs[b], csems[b]).start()
    pltpu.make_async_copy(
        emb_hbm.at[pl.ds(pl.multiple_of(i0, RW), SPAN)], span, wsem).wait()

    def chunk_group(cg, carry):
        for b in range(NBUF):
            c = cg * NBUF + b
            pltpu.make_async_copy(x_chunk(c), xbs[b], csems[b]).wait()

            @pl.when(cg >= 1)
            def _():
                pltpu.make_async_copy(
                    ress[b], out_block(c - NBUF), ssems[b]).wait()

            xb, res = xbs[b], ress[b]
            base = (S - 1) - c * W
            for m in range(W):
                xrow = [xb[m, pl.ds(k * L, L)] for k in range(D // L)]
                for i_r in range(RW):
                    o = base + i_r - m
                    for k in range(D // L):
                        sl = pl.ds(k * L, L)
                        res[i_r, m, sl] = xrow[k] + span[o, sl]

            pltpu.make_async_copy(res, out_block(c), ssems[b]).start()

            @pl.when(cg < NCH // NBUF - 1)
            def _():
                pltpu.make_async_copy(x_chunk(c + NBUF), xbs[b], csems[b]).start()
        return carry

    lax.fori_loop(0, NCH // NBUF, chunk_group, 0)

    for b in range(NBUF):
        pltpu.make_async_copy(
            ress[b], out_block(NCH - NBUF + b), ssems[b]).wait()


_sc_call = functools.partial(
    pl.kernel,
    mesh=plsc.VectorSubcoreMesh(core_axis_name="c", subcore_axis_name="s"),
    out_type=jax.ShapeDtypeStruct((S, S, D), jnp.float32),
    scratch_types=(
        [pltpu.VMEM((SPAN, D), jnp.float32)]
        + [pltpu.VMEM((W, D), jnp.float32) for _ in range(NBUF)]
        + [pltpu.VMEM((RW, W, D), jnp.float32) for _ in range(NBUF)]
        + [pltpu.SemaphoreType.DMA for _ in range(1 + 2 * NBUF)]
    ),
)(_sc_body)


def kernel(x, emb_table):
    emb_pad = jnp.concatenate(
        [emb_table, jnp.zeros((1, D), emb_table.dtype)], axis=0)
    return _sc_call(emb_pad, x[0])


# speedup vs baseline: 3.1259x; 3.1259x over previous
import functools

import jax
import jax.numpy as jnp
from jax import lax
from jax.experimental import pallas as pl
from jax.experimental.pallas import tpu as pltpu
from jax.experimental.pallas import tpu_sc as plsc

S = 512
D = 128
T = 2 * S - 1
NC = 2
NS = 16
NW = NC * NS
RW = S // NW
W = 8
NCH = S // W
WIN = W + RW
L = 16
NBUF = 2


def _win_lo(i0, c):
    return pl.multiple_of(i0 + (S - 1) - c * W - (W - 1), W)


def _sc_body(emb_hbm, x_hbm, out_hbm, *refs):
    wins = refs[0:NBUF]
    xbs = refs[NBUF:2 * NBUF]
    ress = refs[2 * NBUF:3 * NBUF]
    csems = refs[3 * NBUF:4 * NBUF]
    ssems = refs[4 * NBUF:5 * NBUF]

    wid = lax.axis_index("s") * NC + lax.axis_index("c")
    i0 = wid * RW

    def issue_copies(c, b):
        pltpu.make_async_copy(
            emb_hbm.at[pl.ds(_win_lo(i0, c), WIN)], wins[b], csems[b]).start()
        pltpu.make_async_copy(
            x_hbm.at[pl.ds(pl.multiple_of(c * W, W), W)], xbs[b], csems[b]).start()

    def wait_copies(c, b):
        pltpu.make_async_copy(
            emb_hbm.at[pl.ds(_win_lo(i0, c), WIN)], wins[b], csems[b]).wait()
        pltpu.make_async_copy(
            x_hbm.at[pl.ds(pl.multiple_of(c * W, W), W)], xbs[b], csems[b]).wait()

    def out_block(c):
        return out_hbm.at[pl.ds(pl.multiple_of(i0, RW), RW),
                          pl.ds(pl.multiple_of(c * W, W), W)]

    for b in range(NBUF):
        issue_copies(b, b)

    def chunk_group(cg, carry):
        for b in range(NBUF):
            c = cg * NBUF + b
            wait_copies(c, b)

            @pl.when(cg >= 1)
            def _():
                pltpu.make_async_copy(
                    ress[b], out_block(c - NBUF), ssems[b]).wait()

            win, xb, res = wins[b], xbs[b], ress[b]

            for m in range(W):
                xrow = [xb[m, pl.ds(k * L, L)] for k in range(D // L)]
                for i_r in range(RW):
                    o = (W - 1) + i_r - m
                    for k in range(D // L):
                        sl = pl.ds(k * L, L)
                        res[i_r, m, sl] = xrow[k] + win[o, sl]

            pltpu.make_async_copy(res, out_block(c), ssems[b]).start()

            @pl.when(cg < NCH // NBUF - 1)
            def _():
                issue_copies(c + NBUF, b)
        return carry

    lax.fori_loop(0, NCH // NBUF, chunk_group, 0)

    for b in range(NBUF):
        pltpu.make_async_copy(
            ress[b], out_block(NCH - NBUF + b), ssems[b]).wait()


_sc_call = functools.partial(
    pl.kernel,
    mesh=plsc.VectorSubcoreMesh(core_axis_name="c", subcore_axis_name="s"),
    out_type=jax.ShapeDtypeStruct((S, S, D), jnp.float32),
    scratch_types=(
        [pltpu.VMEM((WIN, D), jnp.float32) for _ in range(NBUF)]
        + [pltpu.VMEM((W, D), jnp.float32) for _ in range(NBUF)]
        + [pltpu.VMEM((RW, W, D), jnp.float32) for _ in range(NBUF)]
        + [pltpu.SemaphoreType.DMA for _ in range(2 * NBUF)]
    ),
)(_sc_body)


def kernel(x, emb_table):
    emb_pad = jnp.concatenate(
        [emb_table, jnp.zeros((1, D), emb_table.dtype)], axis=0)
    return _sc_call(emb_pad, x[0])


# speedup vs baseline: 3.6141x; 1.1562x over previous
import functools

import jax
import jax.numpy as jnp
from jax import lax
from jax.experimental import pallas as pl
from jax.experimental.pallas import tpu as pltpu
from jax.experimental.pallas import tpu_sc as plsc

S = 512
D = 128
T = 2 * S - 1
NC = 2
NS = 16
NW = NC * NS
RW = S // NW
W = 8
NCH = S // W
WIN = W + RW
L = 16
NBUF = 2


def _win_lo(i0, c):
    return pl.multiple_of(i0 + (S - 1) - c * W - (W - 1), W)


def _sc_body(emb_hbm, x_hbm, out_hbm, sh_tab, sh_x, bt, bx, stsem, *refs):
    wins = refs[0:NBUF]
    xbs = refs[NBUF:2 * NBUF]
    ress = refs[2 * NBUF:3 * NBUF]
    csems = refs[3 * NBUF:4 * NBUF]
    ssems = refs[4 * NBUF:5 * NBUF]

    wid = lax.axis_index("s") * NC + lax.axis_index("c")
    i0 = wid * RW

    sid = lax.axis_index("s")
    tp = (T + 1) // NS
    xp = S // NS
    t_off = pl.multiple_of(sid * tp, 8)
    x_off = pl.multiple_of(sid * xp, 8)
    pltpu.make_async_copy(emb_hbm.at[pl.ds(t_off, tp)], bt, stsem).start()
    pltpu.make_async_copy(x_hbm.at[pl.ds(x_off, xp)], bx, stsem).start()
    pltpu.make_async_copy(emb_hbm.at[pl.ds(t_off, tp)], bt, stsem).wait()
    pltpu.make_async_copy(x_hbm.at[pl.ds(x_off, xp)], bx, stsem).wait()
    pltpu.sync_copy(bt, sh_tab.at[pl.ds(t_off, tp)])
    pltpu.sync_copy(bx, sh_x.at[pl.ds(x_off, xp)])
    plsc.subcore_barrier()

    def issue_copies(c, b):
        pltpu.make_async_copy(
            sh_tab.at[pl.ds(_win_lo(i0, c), WIN)], wins[b], csems[b]).start()
        pltpu.make_async_copy(
            sh_x.at[pl.ds(pl.multiple_of(c * W, W), W)], xbs[b], csems[b]).start()

    def wait_copies(c, b):
        pltpu.make_async_copy(
            sh_tab.at[pl.ds(_win_lo(i0, c), WIN)], wins[b], csems[b]).wait()
        pltpu.make_async_copy(
            sh_x.at[pl.ds(pl.multiple_of(c * W, W), W)], xbs[b], csems[b]).wait()

    def out_block(c):
        return out_hbm.at[pl.ds(pl.multiple_of(i0, RW), RW),
                          pl.ds(pl.multiple_of(c * W, W), W)]

    for b in range(NBUF):
        issue_copies(b, b)

    def chunk_group(cg, carry):
        for b in range(NBUF):
            c = cg * NBUF + b
            wait_copies(c, b)

            @pl.when(cg >= 1)
            def _():
                pltpu.make_async_copy(
                    ress[b], out_block(c - NBUF), ssems[b]).wait()

            win, xb, res = wins[b], xbs[b], ress[b]

            for m in range(W):
                xrow = [xb[m, pl.ds(k * L, L)] for k in range(D // L)]
                for i_r in range(RW):
                    o = (W - 1) + i_r - m
                    for k in range(D // L):
                        sl = pl.ds(k * L, L)
                        res[i_r, m, sl] = xrow[k] + win[o, sl]

            pltpu.make_async_copy(res, out_block(c), ssems[b]).start()

            @pl.when(cg < NCH // NBUF - 1)
            def _():
                issue_copies(c + NBUF, b)
        return carry

    lax.fori_loop(0, NCH // NBUF, chunk_group, 0)

    for b in range(NBUF):
        pltpu.make_async_copy(
            ress[b], out_block(NCH - NBUF + b), ssems[b]).wait()


_sc_call = functools.partial(
    pl.kernel,
    mesh=plsc.VectorSubcoreMesh(core_axis_name="c", subcore_axis_name="s"),
    out_type=jax.ShapeDtypeStruct((S, S, D), jnp.float32),
    scratch_types=(
        [pltpu.VMEM_SHARED((T + 1, D), jnp.float32),
         pltpu.VMEM_SHARED((S, D), jnp.float32),
         pltpu.VMEM(((T + 1) // NS, D), jnp.float32),
         pltpu.VMEM((S // NS, D), jnp.float32),
         pltpu.SemaphoreType.DMA]
        + [pltpu.VMEM((WIN, D), jnp.float32) for _ in range(NBUF)]
        + [pltpu.VMEM((W, D), jnp.float32) for _ in range(NBUF)]
        + [pltpu.VMEM((RW, W, D), jnp.float32) for _ in range(NBUF)]
        + [pltpu.SemaphoreType.DMA for _ in range(2 * NBUF)]
    ),
)(_sc_body)


def kernel(x, emb_table):
    emb_pad = jnp.concatenate(
        [emb_table, jnp.zeros((1, D), emb_table.dtype)], axis=0)
    return _sc_call(emb_pad, x[0])
